# Initial kernel scaffold; baseline (speedup 1.0000x reference)
import jax
import jax.numpy as jnp
from jax.experimental import pallas as pl


def kernel(x_attr, node_assign, W_pre, b_pre, emb_W, emb_b, W_ops, b_ops, W_res1, b_res1, W_res2, b_res2):
    raise NotImplementedError("write your pallas kernel here")


# speedup vs baseline: 3.7235x; 3.7235x over previous
import functools

import jax
import jax.numpy as jnp
from jax.experimental import pallas as pl

N_TOTAL = 50000
N_ATTR = 10000
D_IN = 512
D_HID = 256
K = 8

TILE_A = 1000
TILE_U = 1000


def _elu(x):
    return jnp.where(x > 0, x, jnp.exp(jnp.minimum(x, 0.0)) - 1.0)


def _attr_body(x_ref, a_ref, wpre_ref, bpre_ref, wops_ref, bops_ref,
               w1_ref, b1_ref, w2_ref, b2_ref, out_ref):
    x = x_ref[...]
    h_tr = jnp.dot(x, wpre_ref[...], preferred_element_type=jnp.float32)
    h_tr = h_tr + bpre_ref[...]
    a = a_ref[0, 0, :][:, None]
    acc = jnp.zeros((TILE_A, D_HID), dtype=jnp.float32)
    for k in range(1, K):
        o = jnp.dot(h_tr, wops_ref[k - 1], preferred_element_type=jnp.float32)
        o = _elu(o + bops_ref[k - 1][None, :])
        acc = acc + jnp.where(a == k, o, 0.0)
    t = _elu(jnp.dot(acc, w1_ref[...], preferred_element_type=jnp.float32)
             + b1_ref[...])
    res = _elu(jnp.dot(t, w2_ref[...], preferred_element_type=jnp.float32)
               + b2_ref[...])
    out_ref[...] = _elu(acc + res) + h_tr


def _unattr_body(e_ref, a_ref, embb_ref, bops_ref,
                 w1_ref, b1_ref, w2_ref, b2_ref, out_ref):
    a = a_ref[0, 0, :][:, None]
    e = e_ref[...] + embb_ref[...]
    h_att = jnp.where(a == 0, e, 0.0)
    for k in range(1, K):
        c = _elu(bops_ref[k - 1][None, :])
        h_att = h_att + jnp.where(a == k, c, 0.0)
    t = _elu(jnp.dot(h_att, w1_ref[...], preferred_element_type=jnp.float32)
             + b1_ref[...])
    res = _elu(jnp.dot(t, w2_ref[...], preferred_element_type=jnp.float32)
               + b2_ref[...])
    out_ref[...] = _elu(h_att + res)


@functools.partial(jax.jit, static_argnames=("interpret",))
def kernel(x_attr, node_assign, W_pre, b_pre, emb_W, emb_b, W_ops, b_ops,
           W_res1, b_res1, W_res2, b_res2, interpret=False):
    node_assign = node_assign.astype(jnp.int32)
    a_attr = node_assign[:N_ATTR].reshape(N_ATTR // TILE_A, 1, TILE_A)
    a_un = node_assign[N_ATTR:].reshape((N_TOTAL - N_ATTR) // TILE_U, 1, TILE_U)
    b_pre2 = b_pre.reshape(1, D_HID)
    emb_b2 = emb_b.reshape(1, D_HID)
    b1_2 = b_res1.reshape(1, 2 * D_HID)
    b2_2 = b_res2.reshape(1, D_HID)

    const_spec = lambda shape: pl.BlockSpec(shape, lambda i: (0,) * len(shape))

    out_a = pl.pallas_call(
        _attr_body,
        grid=(N_ATTR // TILE_A,),
        in_specs=[
            pl.BlockSpec((TILE_A, D_IN), lambda i: (i, 0)),
            pl.BlockSpec((1, 1, TILE_A), lambda i: (i, 0, 0)),
            const_spec((D_IN, D_HID)),
            const_spec((1, D_HID)),
            const_spec((K - 1, D_HID, D_HID)),
            const_spec((K - 1, D_HID)),
            const_spec((D_HID, 2 * D_HID)),
            const_spec((1, 2 * D_HID)),
            const_spec((2 * D_HID, D_HID)),
            const_spec((1, D_HID)),
        ],
        out_specs=pl.BlockSpec((TILE_A, D_HID), lambda i: (i, 0)),
        out_shape=jax.ShapeDtypeStruct((N_ATTR, D_HID), jnp.float32),
        interpret=interpret,
    )(x_attr, a_attr, W_pre, b_pre2, W_ops, b_ops, W_res1, b1_2, W_res2, b2_2)

    out_u = pl.pallas_call(
        _unattr_body,
        grid=((N_TOTAL - N_ATTR) // TILE_U,),
        in_specs=[
            pl.BlockSpec((TILE_U, D_HID), lambda i: (i, 0)),
            pl.BlockSpec((1, 1, TILE_U), lambda i: (i, 0, 0)),
            const_spec((1, D_HID)),
            const_spec((K - 1, D_HID)),
            const_spec((D_HID, 2 * D_HID)),
            const_spec((1, 2 * D_HID)),
            const_spec((2 * D_HID, D_HID)),
            const_spec((1, D_HID)),
        ],
        out_specs=pl.BlockSpec((TILE_U, D_HID), lambda i: (i, 0)),
        out_shape=jax.ShapeDtypeStruct((N_TOTAL - N_ATTR, D_HID), jnp.float32),
        interpret=interpret,
    )(emb_W, a_un, emb_b2, b_ops, W_res1, b1_2, W_res2, b2_2)

    return jnp.concatenate([out_a, out_u], axis=0)
